# baseline (speedup 1.0000x reference)
import functools

import jax
import jax.numpy as jnp
from jax import lax
from jax.experimental import pallas as pl
from jax.experimental.pallas import tpu as pltpu
from jax.experimental.pallas import tpu_sc as plsc

_B = 16384
_S = 50
_D = 32
_NC, _NS = 2, 16
_NW = _NC * _NS
_BT_PER_W = (_B // 128) // _NW
_N_TASK = _S * _BT_PER_W

_mesh = plsc.VectorSubcoreMesh(
    core_axis_name="c", subcore_axis_name="s", num_cores=_NC, num_subcores=_NS
)


@functools.partial(
    pl.kernel,
    out_type=jax.ShapeDtypeStruct((_S, _D // 8, _B // 128, 8, 128), jnp.float32),
    mesh=_mesh,
    compiler_params=pltpu.CompilerParams(use_tc_tiling_on_sc=False, needs_layout_passes=False),
    scratch_types=[
        pltpu.VMEM((_S, 128 * _BT_PER_W), jnp.int32),
        pltpu.VMEM((2, 128, _D), jnp.float32),
        pltpu.VMEM((2, _D, 128), jnp.float32),
        pltpu.SemaphoreType.DMA,
        pltpu.SemaphoreType.DMA,
        pltpu.SemaphoreType.DMA,
        pltpu.SemaphoreType.DMA,
    ],
)
def _gather_kernel(
    idxt_hbm, table_hbm, out_hbm, idx_v, rows_v, trans_v,
    gsem0, gsem1, osem0, osem1
):
    wid = lax.axis_index("s") * _NC + lax.axis_index("c")
    bt0 = wid * _BT_PER_W
    gsems = (gsem0, gsem1)
    osems = (osem0, osem1)

    pltpu.sync_copy(
        idxt_hbm.at[:, pl.ds(wid * (128 * _BT_PER_W), 128 * _BT_PER_W)], idx_v
    )

    lane = lax.iota(jnp.int32, 16)
    row_iota = [lane + g * 16 for g in range(8)]
    col_const = [jnp.full((16,), c, jnp.int32) for c in range(_D)]

    def task_sj(t):
        return t // _BT_PER_W, t % _BT_PER_W

    def fire_g(t, buf):
        s, j = task_sj(t)
        pltpu.async_copy(
            table_hbm.at[idx_v.at[s, pl.ds(j * 128, 128)]],
            rows_v.at[buf],
            gsems[buf],
        )

    def wait_g(t, buf):
        s, j = task_sj(t)
        pltpu.make_async_copy(
            table_hbm.at[idx_v.at[s, pl.ds(j * 128, 128)]],
            rows_v.at[buf],
            gsems[buf],
        ).wait()

    def transpose(buf):
        src = rows_v.at[buf]
        dst = trans_v.at[buf]
        for c in range(_D):
            for g in range(8):
                v = plsc.load_gather(src, [row_iota[g], col_const[c]])
                dst[c, pl.ds(g * 16, 16)] = v

    def write(t, buf):
        s, j = task_sj(t)
        for ct in range(_D // 8):
            pltpu.async_copy(
                trans_v.at[buf, pl.ds(ct * 8, 8)],
                out_hbm.at[s, ct, bt0 + j],
                osems[buf],
            )

    def wait_write(t, buf):
        s, j = task_sj(t)
        for ct in range(_D // 8):
            pltpu.make_async_copy(
                trans_v.at[buf, pl.ds(ct * 8, 8)],
                out_hbm.at[s, ct, bt0 + j],
                osems[buf],
            ).wait()

    fire_g(0, 0)
    fire_g(1, 1)
    for t0 in (0, 1):
        wait_g(t0, t0)
        transpose(t0)
        write(t0, t0)
        fire_g(t0 + 2, t0)

    def body(i, _):
        for k in range(2):
            t = 2 + 2 * i + k
            wait_g(t, k)
            wait_write(t - 2, k)
            transpose(k)
            write(t, k)
            fire_g(t + 2, k)
        return 0

    lax.fori_loop(0, (_N_TASK - 4) // 2, body, 0)

    for t0 in (_N_TASK - 2, _N_TASK - 1):
        k = t0 % 2
        wait_g(t0, k)
        wait_write(t0 - 2, k)
        transpose(k)
        write(t0, k)
        wait_write(t0, k)


def kernel(input_, weight):
    idxt = input_.T.astype(jnp.int32)
    out5 = _gather_kernel(idxt, weight)
    return jnp.transpose(out5, (2, 4, 0, 1, 3)).reshape(_B, _S, _D)


# speedup vs baseline: 1.0742x; 1.0742x over previous
import functools

import jax
import jax.numpy as jnp
from jax import lax
from jax.experimental import pallas as pl
from jax.experimental.pallas import tpu as pltpu
from jax.experimental.pallas import tpu_sc as plsc

_B = 16384
_S = 50
_D = 32
_NC, _NS = 2, 16
_NW = _NC * _NS
_BT_PER_W = (_B // 128) // _NW
_N_TASK = _S * _BT_PER_W

_mesh = plsc.VectorSubcoreMesh(
    core_axis_name="c", subcore_axis_name="s", num_cores=_NC, num_subcores=_NS
)


@functools.partial(
    pl.kernel,
    out_type=jax.ShapeDtypeStruct((_S, _D // 8, _B // 128, 8, 128), jnp.float32),
    mesh=_mesh,
    compiler_params=pltpu.CompilerParams(use_tc_tiling_on_sc=False, needs_layout_passes=False),
    scratch_types=[
        pltpu.VMEM((_S, 128 * _BT_PER_W), jnp.int32),
        pltpu.VMEM((2, 128, _D), jnp.float32),
        pltpu.VMEM((2, _D, 128), jnp.float32),
        pltpu.SemaphoreType.DMA,
        pltpu.SemaphoreType.DMA,
        pltpu.SemaphoreType.DMA,
        pltpu.SemaphoreType.DMA,
    ],
)
def _gather_kernel(
    idxt_hbm, table_hbm, out_hbm, idx_v, rows_v, trans_v,
    gsem0, gsem1, osem0, osem1
):
    wid = lax.axis_index("s") * _NC + lax.axis_index("c")
    bt0 = wid * _BT_PER_W
    gsems = (gsem0, gsem1)
    osems = (osem0, osem1)

    pltpu.sync_copy(
        idxt_hbm.at[:, pl.ds(wid * (128 * _BT_PER_W), 128 * _BT_PER_W)], idx_v
    )

    lane = lax.iota(jnp.int32, 16)
    row_iota = [lane + g * 16 for g in range(8)]

    def task_sj(t):
        return t // _BT_PER_W, t % _BT_PER_W

    def fire_g(t, buf):
        s, j = task_sj(t)
        pltpu.async_copy(
            table_hbm.at[idx_v.at[s, pl.ds(j * 128, 128)]],
            rows_v.at[buf],
            gsems[buf],
        )

    def wait_g(t, buf):
        s, j = task_sj(t)
        pltpu.make_async_copy(
            table_hbm.at[idx_v.at[s, pl.ds(j * 128, 128)]],
            rows_v.at[buf],
            gsems[buf],
        ).wait()

    def transpose(buf):
        src = rows_v.at[buf]
        dst = trans_v.at[buf]

        def tbody(c, _):
            cc = jnp.full((16,), c, jnp.int32)
            for g in range(8):
                v = plsc.load_gather(src, [row_iota[g], cc])
                dst[c, pl.ds(g * 16, 16)] = v
            return 0

        lax.fori_loop(0, _D, tbody, 0)

    def write(t, buf):
        s, j = task_sj(t)
        for ct in range(_D // 8):
            pltpu.async_copy(
                trans_v.at[buf, pl.ds(ct * 8, 8)],
                out_hbm.at[s, ct, bt0 + j],
                osems[buf],
            )

    def wait_write(t, buf):
        s, j = task_sj(t)
        for ct in range(_D // 8):
            pltpu.make_async_copy(
                trans_v.at[buf, pl.ds(ct * 8, 8)],
                out_hbm.at[s, ct, bt0 + j],
                osems[buf],
            ).wait()

    fire_g(0, 0)
    fire_g(1, 1)
    for t0 in (0, 1):
        wait_g(t0, t0)
        transpose(t0)
        write(t0, t0)
        fire_g(t0 + 2, t0)

    def body(i, _):
        for k in range(2):
            t = 2 + 2 * i + k
            wait_g(t, k)
            wait_write(t - 2, k)
            transpose(k)
            write(t, k)
            fire_g(t + 2, k)
        return 0

    lax.fori_loop(0, (_N_TASK - 4) // 2, body, 0)

    for t0 in (_N_TASK - 2, _N_TASK - 1):
        k = t0 % 2
        wait_g(t0, k)
        wait_write(t0 - 2, k)
        transpose(k)
        write(t0, k)
        wait_write(t0, k)


def kernel(input_, weight):
    idxt = input_.T.astype(jnp.int32)
    out5 = _gather_kernel(idxt, weight)
    return jnp.transpose(out5, (2, 4, 0, 1, 3)).reshape(_B, _S, _D)


# speedup vs baseline: 1.8107x; 1.6857x over previous
import functools

import jax
import jax.numpy as jnp
from jax import lax
from jax.experimental import pallas as pl
from jax.experimental.pallas import tpu as pltpu
from jax.experimental.pallas import tpu_sc as plsc

_B = 16384
_S = 50
_D = 32
_NC, _NS = 2, 16
_NW = _NC * _NS
_BT_PER_W = (_B // 128) // _NW
_N_TASK = _S * _BT_PER_W

_mesh = plsc.VectorSubcoreMesh(
    core_axis_name="c", subcore_axis_name="s", num_cores=_NC, num_subcores=_NS
)


@functools.partial(
    pl.kernel,
    out_type=jax.ShapeDtypeStruct((_S, _D // 8, _B // 128, 8, 128), jnp.float32),
    mesh=_mesh,
    compiler_params=pltpu.CompilerParams(use_tc_tiling_on_sc=False, needs_layout_passes=False),
    scratch_types=[
        pltpu.VMEM((_S, 128 * _BT_PER_W), jnp.int32),
        pltpu.VMEM((2, 128, _D), jnp.float32),
        pltpu.VMEM((2, _D, 129), jnp.float32),
        pltpu.SemaphoreType.DMA,
        pltpu.SemaphoreType.DMA,
        pltpu.SemaphoreType.DMA,
        pltpu.SemaphoreType.DMA,
    ],
)
def _gather_kernel(
    idxt_hbm, table_hbm, out_hbm, idx_v, rows_v, trans_v,
    gsem0, gsem1, osem0, osem1
):
    wid = lax.axis_index("s") * _NC + lax.axis_index("c")
    bt0 = wid * _BT_PER_W
    gsems = (gsem0, gsem1)
    osems = (osem0, osem1)

    pltpu.sync_copy(
        idxt_hbm.at[:, pl.ds(wid * (128 * _BT_PER_W), 128 * _BT_PER_W)], idx_v
    )

    lane = lax.iota(jnp.int32, 16)
    ci0 = lane
    ci1 = lane + 16

    def task_sj(t):
        return t // _BT_PER_W, t % _BT_PER_W

    def fire_g(t, buf):
        s, j = task_sj(t)
        pltpu.async_copy(
            table_hbm.at[idx_v.at[s, pl.ds(j * 128, 128)]],
            rows_v.at[buf],
            gsems[buf],
        )

    def wait_g(t, buf):
        s, j = task_sj(t)
        pltpu.make_async_copy(
            table_hbm.at[idx_v.at[s, pl.ds(j * 128, 128)]],
            rows_v.at[buf],
            gsems[buf],
        ).wait()

    def transpose(buf):
        src = rows_v.at[buf]
        dst = trans_v.at[buf]

        def tbody(b4, _):
            for u in range(4):
                bi = b4 * 4 + u
                sp = jnp.full((16,), 0, jnp.int32) + bi
                v0 = src[bi, pl.ds(0, 16)]
                v1 = src[bi, pl.ds(16, 16)]
                plsc.store_scatter(dst, [ci0, sp], v0)
                plsc.store_scatter(dst, [ci1, sp], v1)
            return 0

        lax.fori_loop(0, 32, tbody, 0)

    def write(t, buf):
        s, j = task_sj(t)
        for c in range(_D):
            pltpu.async_copy(
                trans_v.at[buf, c, pl.ds(0, 128)],
                out_hbm.at[s, c // 8, bt0 + j, c % 8],
                osems[buf],
            )

    def wait_write(t, buf):
        s, j = task_sj(t)
        for ct in range(_D // 8):
            pltpu.make_async_copy(
                trans_v.at[buf, pl.ds(ct * 8, 8), pl.ds(0, 128)],
                out_hbm.at[s, ct, bt0 + j],
                osems[buf],
            ).wait()

    fire_g(0, 0)
    fire_g(1, 1)
    for t0 in (0, 1):
        wait_g(t0, t0)
        transpose(t0)
        write(t0, t0)
        fire_g(t0 + 2, t0)

    def body(i, _):
        for k in range(2):
            t = 2 + 2 * i + k
            wait_g(t, k)
            wait_write(t - 2, k)
            transpose(k)
            write(t, k)
            fire_g(t + 2, k)
        return 0

    lax.fori_loop(0, (_N_TASK - 4) // 2, body, 0)

    for t0 in (_N_TASK - 2, _N_TASK - 1):
        k = t0 % 2
        wait_g(t0, k)
        wait_write(t0 - 2, k)
        transpose(k)
        write(t0, k)
        wait_write(t0, k)


def kernel(input_, weight):
    idxt = input_.T.astype(jnp.int32)
    out5 = _gather_kernel(idxt, weight)
    return jnp.transpose(out5, (2, 4, 0, 1, 3)).reshape(_B, _S, _D)
